# baseline (speedup 1.0000x reference)
import math

import jax
import jax.numpy as jnp
from jax.experimental import pallas as pl
from jax.experimental.pallas import tpu as pltpu

_LAMDA = 1.5
_ALPHA = 0.7


def _bdot(a, b):
    return jnp.dot(
        a.astype(jnp.bfloat16),
        b.astype(jnp.bfloat16),
        preferred_element_type=jnp.float32,
    )


_TILES = 4


def _fused_body(x_ref, adj_ref, w0_ref, b0_ref, w1_ref, w2_ref, w3_ref, o_ref):
    n = x_ref.shape[1]
    t = n // _TILES
    w0 = w0_ref[...]
    b0 = b0_ref[...]
    hs = []
    for r in range(_TILES):
        hr = _bdot(x_ref[0, r * t:(r + 1) * t, :], w0)
        hs.append(jnp.maximum(hr + b0, 0.0))
    h0s = list(hs)
    adj = adj_ref[...].astype(jnp.bfloat16)
    for i, w_ref in enumerate((w1_ref, w2_ref, w3_ref), start=1):
        theta = min(1.0, math.log(_LAMDA / i + 1.0))
        w = w_ref[...]
        hb = jnp.concatenate([h.astype(jnp.bfloat16) for h in hs], axis=0)
        new_hs = []
        for r in range(_TILES):
            hi = jnp.dot(adj[r * t:(r + 1) * t, :], hb, preferred_element_type=jnp.float32)
            support = (1.0 - _ALPHA) * hi + _ALPHA * h0s[r]
            out = theta * _bdot(support, w)
            out = out + (1.0 - theta) * support + hs[r]
            new_hs.append(jnp.maximum(out, 0.0))
        hs = new_hs
    for r in range(_TILES):
        o_ref[0, r * t:(r + 1) * t, :] = hs[r]


def kernel(x, adj, W0, b0, W1, W2, W3):
    B, N, F = x.shape
    H = W0.shape[1]
    b0_2d = b0.reshape(1, H)

    return pl.pallas_call(
        _fused_body,
        grid=(B,),
        in_specs=[
            pl.BlockSpec((1, N, F), lambda b: (b, 0, 0)),
            pl.BlockSpec((N, N), lambda b: (0, 0)),
            pl.BlockSpec((F, H), lambda b: (0, 0)),
            pl.BlockSpec((1, H), lambda b: (0, 0)),
            pl.BlockSpec((H, H), lambda b: (0, 0)),
            pl.BlockSpec((H, H), lambda b: (0, 0)),
            pl.BlockSpec((H, H), lambda b: (0, 0)),
        ],
        out_specs=pl.BlockSpec((1, N, H), lambda b: (b, 0, 0)),
        out_shape=jax.ShapeDtypeStruct((B, N, H), jnp.float32),
        compiler_params=pltpu.CompilerParams(
            dimension_semantics=("parallel",),
        ),
    )(x, adj, W0, b0_2d, W1, W2, W3)


# speedup vs baseline: 1.1442x; 1.1442x over previous
import math

import jax
import jax.numpy as jnp
from jax.experimental import pallas as pl
from jax.experimental.pallas import tpu as pltpu

_LAMDA = 1.5
_ALPHA = 0.7


def _fused_body(x_ref, adj_ref, w0_ref, b0_ref, w1_ref, w2_ref, w3_ref, o_ref,
                adjb_ref, w0b_ref, w1b_ref, w2b_ref, w3b_ref):
    @pl.when(pl.program_id(0) == 0)
    def _cast_invariants():
        adjb_ref[...] = adj_ref[...].astype(jnp.bfloat16)
        w0b_ref[...] = w0_ref[...].astype(jnp.bfloat16)
        w1b_ref[...] = w1_ref[...].astype(jnp.bfloat16)
        w2b_ref[...] = w2_ref[...].astype(jnp.bfloat16)
        w3b_ref[...] = w3_ref[...].astype(jnp.bfloat16)

    x = x_ref[0].astype(jnp.bfloat16)
    h = jnp.dot(x, w0b_ref[...], preferred_element_type=jnp.float32)
    h = jnp.maximum(h + b0_ref[...], 0.0)
    h0 = h
    adj = adjb_ref[...]
    for i, wb_ref in enumerate((w1b_ref, w2b_ref, w3b_ref), start=1):
        theta = min(1.0, math.log(_LAMDA / i + 1.0))
        hi = jnp.dot(adj, h.astype(jnp.bfloat16), preferred_element_type=jnp.float32)
        support = (1.0 - _ALPHA) * hi + _ALPHA * h0
        out = theta * jnp.dot(
            support.astype(jnp.bfloat16), wb_ref[...], preferred_element_type=jnp.float32
        )
        out = out + (1.0 - theta) * support + h
        h = jnp.maximum(out, 0.0)
    o_ref[0] = h


def kernel(x, adj, W0, b0, W1, W2, W3):
    B, N, F = x.shape
    H = W0.shape[1]
    b0_2d = b0.reshape(1, H)

    return pl.pallas_call(
        _fused_body,
        grid=(B,),
        in_specs=[
            pl.BlockSpec((1, N, F), lambda b: (b, 0, 0)),
            pl.BlockSpec((N, N), lambda b: (0, 0)),
            pl.BlockSpec((F, H), lambda b: (0, 0)),
            pl.BlockSpec((1, H), lambda b: (0, 0)),
            pl.BlockSpec((H, H), lambda b: (0, 0)),
            pl.BlockSpec((H, H), lambda b: (0, 0)),
            pl.BlockSpec((H, H), lambda b: (0, 0)),
        ],
        out_specs=pl.BlockSpec((1, N, H), lambda b: (b, 0, 0)),
        out_shape=jax.ShapeDtypeStruct((B, N, H), jnp.float32),
        scratch_shapes=[
            pltpu.VMEM((N, N), jnp.bfloat16),
            pltpu.VMEM((F, H), jnp.bfloat16),
            pltpu.VMEM((H, H), jnp.bfloat16),
            pltpu.VMEM((H, H), jnp.bfloat16),
            pltpu.VMEM((H, H), jnp.bfloat16),
        ],
    )(x, adj, W0, b0_2d, W1, W2, W3)
